# baseline (speedup 1.0000x reference)
import functools

import jax
import jax.numpy as jnp
from jax.experimental import pallas as pl
from jax.experimental.pallas import tpu as pltpu

SLOPE = 0.01
N_BLK = 1000


def _leaky(x):
    return jnp.maximum(x, SLOPE * x)


def _body(n_total, f_ref, x_ref, oh_ref, wc_ref, bc_ref, wa_ref, ba_ref,
          out_ref):
    i = pl.program_id(0)
    nbrs, t_types, nb, d = f_ref.shape
    out = wa_ref.shape[1]

    xb = x_ref[...].astype(jnp.bfloat16)
    acc = jnp.zeros((nb, out), jnp.float32)
    sel = jnp.zeros((nb, d), jnp.float32)
    for t in range(t_types):
        wct = wc_ref[t]
        bct = bc_ref[t]
        s = jnp.zeros((nb, d), jnp.float32)
        c = jnp.zeros((nb, 1), jnp.float32)
        for k in range(nbrs):
            fk = f_ref[k, t]
            z = jax.lax.dot_general(
                fk, wct, (((1,), (0,)), ((), ())),
                preferred_element_type=jnp.float32) + bct
            emb = _leaky(z)
            s += emb
            hi = jnp.max(z, axis=1, keepdims=True)
            lo = jnp.min(z, axis=1, keepdims=True)
            c += ((hi > 0) | (lo < 0)).astype(jnp.float32)
        swa = jax.lax.dot_general(
            s.astype(jnp.bfloat16), wa_ref[t * d:(t + 1) * d, :],
            (((1,), (0,)), ((), ())), preferred_element_type=jnp.float32)
        acc += swa * (1.0 / c)
        y = jax.lax.dot_general(
            xb, wct, (((1,), (0,)), ((), ())),
            preferred_element_type=jnp.float32)
        sel += y * oh_ref[:, t][:, None]
    sel += jax.lax.dot_general(
        oh_ref[...].astype(jnp.bfloat16), bc_ref[...].astype(jnp.bfloat16),
        (((1,), (0,)), ((), ())), preferred_element_type=jnp.float32)
    node_self = _leaky(sel)
    acc += jax.lax.dot_general(
        node_self.astype(jnp.bfloat16), wa_ref[t_types * d:(t_types + 1) * d, :],
        (((1,), (0,)), ((), ())), preferred_element_type=jnp.float32)
    p = jax.nn.sigmoid(acc + ba_ref[...])
    part = jnp.sum(p, axis=0, keepdims=True) * (1.0 / n_total)

    @pl.when(i == 0)
    def _():
        out_ref[...] = jnp.zeros_like(out_ref)

    out_ref[...] += part


def kernel(x_node_feature, x_graph_het_feature, graph_node_types,
           W_content, b_content, W_agg, b_agg):
    n, d = x_node_feature.shape
    t_types, _, nbrs, _ = x_graph_het_feature.shape
    out = W_agg.shape[0]

    n_blk = N_BLK
    grid = n // n_blk

    fT = jnp.transpose(x_graph_het_feature, (2, 0, 1, 3)
                       ).astype(jnp.bfloat16)
    wc_t = jnp.transpose(W_content, (0, 2, 1)).astype(jnp.bfloat16)
    wa_t = jnp.transpose(W_agg, (1, 0)).astype(jnp.bfloat16)
    onehot = (graph_node_types[:, None] == jnp.arange(t_types)[None, :]
              ).astype(jnp.float32)
    ba = b_agg.reshape(1, out)

    res = pl.pallas_call(
        functools.partial(_body, n),
        grid=(grid,),
        in_specs=[
            pl.BlockSpec((nbrs, t_types, n_blk, d), lambda i: (0, 0, i, 0)),
            pl.BlockSpec((n_blk, d), lambda i: (i, 0)),
            pl.BlockSpec((n_blk, t_types), lambda i: (i, 0)),
            pl.BlockSpec((t_types, d, d), lambda i: (0, 0, 0)),
            pl.BlockSpec((t_types, d), lambda i: (0, 0)),
            pl.BlockSpec(((1 + t_types) * d, out), lambda i: (0, 0)),
            pl.BlockSpec((1, out), lambda i: (0, 0)),
        ],
        out_specs=pl.BlockSpec((1, out), lambda i: (0, 0)),
        out_shape=jax.ShapeDtypeStruct((1, out), jnp.float32),
    )(fT, x_node_feature, onehot, wc_t, b_content, wa_t, ba)
    return res[0]


# speedup vs baseline: 1.0155x; 1.0155x over previous
import functools

import jax
import jax.numpy as jnp
from jax.experimental import pallas as pl
from jax.experimental.pallas import tpu as pltpu

SLOPE = 0.01
N_BLK = 1000


def _leaky(x):
    return jnp.maximum(x, SLOPE * x)


def _body(n_total, f_ref, x_ref, oh_ref, wc_ref, bc_ref, wa_ref, ba_ref,
          out_ref):
    i = pl.program_id(0)
    nbrs, t_types, nb, d = f_ref.shape
    out = wa_ref.shape[1]

    xb = x_ref[...].astype(jnp.bfloat16)
    acc = jnp.zeros((nb, out), jnp.float32)
    sel = jnp.zeros((nb, d), jnp.float32)
    for t in range(t_types):
        wct = wc_ref[t]
        bct = bc_ref[t]
        s = jnp.zeros((nb, d), jnp.float32)
        c = jnp.zeros((nb, 1), jnp.float32)
        for k in range(nbrs):
            fk = f_ref[k, t]
            z = jax.lax.dot_general(
                fk, wct, (((1,), (0,)), ((), ())),
                preferred_element_type=jnp.float32) + bct
            emb = _leaky(z)
            s += emb
            c += (jnp.max(jnp.abs(z), axis=1, keepdims=True) > 0
                  ).astype(jnp.float32)
        swa = jax.lax.dot_general(
            s.astype(jnp.bfloat16), wa_ref[t * d:(t + 1) * d, :],
            (((1,), (0,)), ((), ())), preferred_element_type=jnp.float32)
        acc += swa * (1.0 / c)
        y = jax.lax.dot_general(
            xb, wct, (((1,), (0,)), ((), ())),
            preferred_element_type=jnp.float32)
        sel += y * oh_ref[:, t][:, None]
    sel += jax.lax.dot_general(
        oh_ref[...].astype(jnp.bfloat16), bc_ref[...].astype(jnp.bfloat16),
        (((1,), (0,)), ((), ())), preferred_element_type=jnp.float32)
    node_self = _leaky(sel)
    acc += jax.lax.dot_general(
        node_self.astype(jnp.bfloat16), wa_ref[t_types * d:(t_types + 1) * d, :],
        (((1,), (0,)), ((), ())), preferred_element_type=jnp.float32)
    p = jax.nn.sigmoid(acc + ba_ref[...])
    part = jnp.sum(p, axis=0, keepdims=True) * (1.0 / n_total)

    @pl.when(i == 0)
    def _():
        out_ref[...] = jnp.zeros_like(out_ref)

    out_ref[...] += part


def kernel(x_node_feature, x_graph_het_feature, graph_node_types,
           W_content, b_content, W_agg, b_agg):
    n, d = x_node_feature.shape
    t_types, _, nbrs, _ = x_graph_het_feature.shape
    out = W_agg.shape[0]

    n_blk = N_BLK
    grid = n // n_blk

    fT = jnp.transpose(x_graph_het_feature, (2, 0, 1, 3)
                       ).astype(jnp.bfloat16)
    wc_t = jnp.transpose(W_content, (0, 2, 1)).astype(jnp.bfloat16)
    wa_t = jnp.transpose(W_agg, (1, 0)).astype(jnp.bfloat16)
    onehot = (graph_node_types[:, None] == jnp.arange(t_types)[None, :]
              ).astype(jnp.float32)
    ba = b_agg.reshape(1, out)

    res = pl.pallas_call(
        functools.partial(_body, n),
        grid=(grid,),
        in_specs=[
            pl.BlockSpec((nbrs, t_types, n_blk, d), lambda i: (0, 0, i, 0)),
            pl.BlockSpec((n_blk, d), lambda i: (i, 0)),
            pl.BlockSpec((n_blk, t_types), lambda i: (i, 0)),
            pl.BlockSpec((t_types, d, d), lambda i: (0, 0, 0)),
            pl.BlockSpec((t_types, d), lambda i: (0, 0)),
            pl.BlockSpec(((1 + t_types) * d, out), lambda i: (0, 0)),
            pl.BlockSpec((1, out), lambda i: (0, 0)),
        ],
        out_specs=pl.BlockSpec((1, out), lambda i: (0, 0)),
        out_shape=jax.ShapeDtypeStruct((1, out), jnp.float32),
    )(fT, x_node_feature, onehot, wc_t, b_content, wa_t, ba)
    return res[0]
